# baseline (speedup 1.0000x reference)
import functools

import jax
import jax.numpy as jnp
from jax import lax
from jax.experimental import pallas as pl
from jax.experimental.pallas import tpu as pltpu
from jax.experimental.pallas import tpu_sc as plsc

_NC = 2
_NS = 16
_NW = _NC * _NS
_L = 16


def _sc_mask_call(e0, e1, c0, c1, labels, E):
    N = labels.shape[0]
    CH = E // _NW

    mesh = plsc.VectorSubcoreMesh(core_axis_name="c", subcore_axis_name="s")

    @functools.partial(
        pl.kernel,
        mesh=mesh,
        compiler_params=pltpu.CompilerParams(needs_layout_passes=False),
        out_type=(
            jax.ShapeDtypeStruct((E,), jnp.float32),
            jax.ShapeDtypeStruct((E,), jnp.int32),
            jax.ShapeDtypeStruct((E,), jnp.int32),
            jax.ShapeDtypeStruct((E,), jnp.float32),
            jax.ShapeDtypeStruct((E,), jnp.float32),
        ),
        scratch_types=[
            pltpu.VMEM((N,), jnp.int32),
            pltpu.VMEM((CH,), jnp.int32),
            pltpu.VMEM((CH,), jnp.int32),
            pltpu.VMEM((CH,), jnp.float32),
            pltpu.VMEM((CH,), jnp.float32),
            pltpu.VMEM((CH,), jnp.float32),
            pltpu.SemaphoreType.DMA,
        ],
    )
    def sc_kern(e0_hbm, e1_hbm, c0_hbm, c1_hbm, labels_hbm,
                m_hbm, e0o_hbm, e1o_hbm, c0o_hbm, c1o_hbm,
                labels_v, e0v, e1v, c0v, c1v, mv, sem_col):
        wid = lax.axis_index("s") * _NC + lax.axis_index("c")
        b1 = wid * CH

        h0 = pltpu.async_copy(labels_hbm, labels_v, sem_col)
        h1 = pltpu.async_copy(e0_hbm.at[pl.ds(b1, CH)], e0v, sem_col)
        h2 = pltpu.async_copy(e1_hbm.at[pl.ds(b1, CH)], e1v, sem_col)
        h3 = pltpu.async_copy(c0_hbm.at[pl.ds(b1, CH)], c0v, sem_col)
        h4 = pltpu.async_copy(c1_hbm.at[pl.ds(b1, CH)], c1v, sem_col)
        h0.wait()
        h1.wait()
        h2.wait()
        h3.wait()
        h4.wait()

        def body_mask(i, carry):
            off = pl.multiple_of(i * _L, _L)
            l0 = plsc.load_gather(labels_v, [e0v[pl.ds(off, _L)]])
            l1 = plsc.load_gather(labels_v, [e1v[pl.ds(off, _L)]])
            k = l0 & l1
            kf = k.astype(jnp.float32)
            mv[pl.ds(off, _L)] = kf
            e0v[pl.ds(off, _L)] = e0v[pl.ds(off, _L)] * k
            e1v[pl.ds(off, _L)] = e1v[pl.ds(off, _L)] * k
            c0v[pl.ds(off, _L)] = c0v[pl.ds(off, _L)] * kf
            c1v[pl.ds(off, _L)] = c1v[pl.ds(off, _L)] * kf
            return carry

        lax.fori_loop(0, CH // _L, body_mask, 0)

        hm = pltpu.async_copy(mv, m_hbm.at[pl.ds(b1, CH)], sem_col)
        hc0 = pltpu.async_copy(e0v, e0o_hbm.at[pl.ds(b1, CH)], sem_col)
        hc1 = pltpu.async_copy(e1v, e1o_hbm.at[pl.ds(b1, CH)], sem_col)
        hc2 = pltpu.async_copy(c0v, c0o_hbm.at[pl.ds(b1, CH)], sem_col)
        hc3 = pltpu.async_copy(c1v, c1o_hbm.at[pl.ds(b1, CH)], sem_col)
        hm.wait()
        hc0.wait()
        hc1.wait()
        hc2.wait()
        hc3.wait()

    return sc_kern(e0, e1, c0, c1, labels)


_BT = 1024


def _tc_body(m_ref, x_ref, o_ref):
    m = m_ref[...]
    x = x_ref[...]
    x3 = x.reshape(_BT // 128, 128, x.shape[-1])
    o3 = x3 * m[:, :, None]
    o_ref[...] = o3.reshape(x.shape)


def _tc_mask_call(edge_feat, mask2d, E, D):
    G = pl.cdiv(E, _BT)
    return pl.pallas_call(
        _tc_body,
        grid=(G,),
        in_specs=[
            pl.BlockSpec((_BT // 128, 128), lambda i: (i, 0)),
            pl.BlockSpec((_BT, D), lambda i: (i, 0)),
        ],
        out_specs=pl.BlockSpec((_BT, D), lambda i: (i, 0)),
        out_shape=jax.ShapeDtypeStruct((E, D), jnp.float32),
    )(mask2d, edge_feat)


def kernel(edge_feat, edges, edge_classes, detector_labels):
    E, D = edge_feat.shape
    edges_i = edges.astype(jnp.int32)
    labels = detector_labels.astype(jnp.int32)

    mask, e0o, e1o, c0o, c1o = _sc_mask_call(
        edges_i[:, 0], edges_i[:, 1],
        edge_classes[:, 0], edge_classes[:, 1],
        labels, E,
    )
    feat_out = _tc_mask_call(edge_feat, mask.reshape(E // 128, 128), E, D)

    return (
        feat_out,
        jnp.stack([e0o, e1o], axis=1).astype(edges.dtype),
        jnp.stack([c0o, c1o], axis=1),
    )


# speedup vs baseline: 1.4464x; 1.4464x over previous
import functools

import jax
import jax.numpy as jnp
from jax import lax
from jax.experimental import pallas as pl
from jax.experimental.pallas import tpu as pltpu
from jax.experimental.pallas import tpu_sc as plsc

_NC = 2
_NS = 16
_NW = _NC * _NS
_L = 16
_NB = 5


def _sc_call(e0, e1, c0, c1, feat, labels, E, D):
    N = labels.shape[0]
    CH = E // _NW
    FB = 80
    NFB = CH // FB

    mesh = plsc.VectorSubcoreMesh(core_axis_name="c", subcore_axis_name="s")

    @functools.partial(
        pl.kernel,
        mesh=mesh,
        compiler_params=pltpu.CompilerParams(needs_layout_passes=False),
        out_type=(
            jax.ShapeDtypeStruct((E, D), jnp.float32),
            jax.ShapeDtypeStruct((E,), jnp.int32),
            jax.ShapeDtypeStruct((E,), jnp.int32),
            jax.ShapeDtypeStruct((E,), jnp.float32),
            jax.ShapeDtypeStruct((E,), jnp.float32),
        ),
        scratch_types=[
            pltpu.VMEM((N,), jnp.int32),
            pltpu.VMEM((CH,), jnp.int32),
            pltpu.VMEM((CH,), jnp.int32),
            pltpu.VMEM((CH,), jnp.float32),
            pltpu.VMEM((CH,), jnp.float32),
            pltpu.VMEM((CH,), jnp.float32),
            pltpu.VMEM((_NB, FB, D), jnp.float32),
            pltpu.SemaphoreType.DMA,
            [pltpu.SemaphoreType.DMA] * _NB,
            [pltpu.SemaphoreType.DMA] * _NB,
        ],
    )
    def sc_kern(e0_hbm, e1_hbm, c0_hbm, c1_hbm, feat_hbm, labels_hbm,
                feat_out_hbm, e0o_hbm, e1o_hbm, c0o_hbm, c1o_hbm,
                labels_v, e0v, e1v, c0v, c1v, mv, fv,
                sem_col, sems_in, sems_out):
        wid = lax.axis_index("s") * _NC + lax.axis_index("c")
        b1 = wid * CH

        def in_slice(blk):
            row0 = pl.multiple_of(b1 + blk * FB, 8)
            return feat_hbm.at[pl.ds(row0, FB)]

        def out_slice(blk):
            row0 = pl.multiple_of(b1 + blk * FB, 8)
            return feat_out_hbm.at[pl.ds(row0, FB)]

        h0 = pltpu.async_copy(labels_hbm, labels_v, sem_col)
        h1 = pltpu.async_copy(e0_hbm.at[pl.ds(b1, CH)], e0v, sem_col)
        h2 = pltpu.async_copy(e1_hbm.at[pl.ds(b1, CH)], e1v, sem_col)
        h3 = pltpu.async_copy(c0_hbm.at[pl.ds(b1, CH)], c0v, sem_col)
        h4 = pltpu.async_copy(c1_hbm.at[pl.ds(b1, CH)], c1v, sem_col)

        for b in range(_NB):
            pltpu.async_copy(in_slice(b), fv.at[b], sems_in[b])

        h0.wait()
        h1.wait()
        h2.wait()
        h3.wait()
        h4.wait()

        def body_mask(i, carry):
            off = pl.multiple_of(i * _L, _L)
            l0 = plsc.load_gather(labels_v, [e0v[pl.ds(off, _L)]])
            l1 = plsc.load_gather(labels_v, [e1v[pl.ds(off, _L)]])
            k = l0 & l1
            kf = k.astype(jnp.float32)
            mv[pl.ds(off, _L)] = kf
            e0v[pl.ds(off, _L)] = e0v[pl.ds(off, _L)] * k
            e1v[pl.ds(off, _L)] = e1v[pl.ds(off, _L)] * k
            c0v[pl.ds(off, _L)] = c0v[pl.ds(off, _L)] * kf
            c1v[pl.ds(off, _L)] = c1v[pl.ds(off, _L)] * kf
            return carry

        lax.fori_loop(0, CH // _L, body_mask, 0)

        hc0 = pltpu.async_copy(e0v, e0o_hbm.at[pl.ds(b1, CH)], sem_col)
        hc1 = pltpu.async_copy(e1v, e1o_hbm.at[pl.ds(b1, CH)], sem_col)
        hc2 = pltpu.async_copy(c0v, c0o_hbm.at[pl.ds(b1, CH)], sem_col)
        hc3 = pltpu.async_copy(c1v, c1o_hbm.at[pl.ds(b1, CH)], sem_col)

        def round_body(g, carry):
            blk0 = g * _NB
            for b in range(_NB):
                blk = blk0 + b
                pltpu.make_async_copy(in_slice(blk), fv.at[b], sems_in[b]).wait()

                basev = jnp.full((_L,), blk * FB, jnp.int32)

                def body_row(e, c2):
                    kf = plsc.load_gather(mv, [basev + e])
                    for j in range(D // _L):
                        fv[b, e, pl.ds(j * _L, _L)] = (
                            fv[b, e, pl.ds(j * _L, _L)] * kf
                        )
                    return c2

                lax.fori_loop(0, FB, body_row, 0)
                pltpu.async_copy(fv.at[b], out_slice(blk), sems_out[b])

                rblk = blk + 2
                rb = (b + 2) % _NB

                @pl.when(jnp.logical_and(rblk >= _NB, rblk < NFB))
                def _():
                    pltpu.make_async_copy(
                        fv.at[rb], out_slice(rblk - _NB), sems_out[rb]
                    ).wait()
                    pltpu.async_copy(in_slice(rblk), fv.at[rb], sems_in[rb])

            return carry

        lax.fori_loop(0, NFB // _NB, round_body, 0)

        for b in range(_NB):
            blk = NFB - _NB + b
            pltpu.make_async_copy(fv.at[b], out_slice(blk), sems_out[b]).wait()
        hc0.wait()
        hc1.wait()
        hc2.wait()
        hc3.wait()

    return sc_kern(e0, e1, c0, c1, feat, labels)


def kernel(edge_feat, edges, edge_classes, detector_labels):
    E, D = edge_feat.shape
    edges_i = edges.astype(jnp.int32)
    labels = detector_labels.astype(jnp.int32)

    feat_out, e0o, e1o, c0o, c1o = _sc_call(
        edges_i[:, 0], edges_i[:, 1],
        edge_classes[:, 0], edge_classes[:, 1],
        edge_feat, labels, E, D,
    )

    return (
        feat_out,
        jnp.stack([e0o, e1o], axis=1).astype(edges.dtype),
        jnp.stack([c0o, c1o], axis=1),
    )
